# baseline (speedup 1.0000x reference)
import jax
import jax.numpy as jnp
from jax import lax
from jax.experimental import pallas as pl
from jax.experimental.pallas import tpu as pltpu
from jax.experimental.pallas import tpu_sc as plsc

N = 10000
E = 320000
D = 128
H = 4
C = D // H

ET = E + N
NTILES = 32
BE = 128
CHUNK = 10752
EPAD = CHUNK * NTILES
NBLK = CHUNK // BE
BE1 = CHUNK // 16
NBLK1 = CHUNK // BE1
EW = 8
NPAD = 10240
ROWS_PER_TILE = NPAD // 16

BN = 1000
NG = N // BN


def _phase_a_body(x_ref, w_ref, as_ref, ad_ref, h_ref, asrc_ref, adst_ref,
                  k_ref, ms_ref, md_ref):
    i = pl.program_id(0)
    h = jnp.dot(x_ref[...], w_ref[...], preferred_element_type=jnp.float32)
    h_ref[...] = h
    asrc = jnp.dot(h, as_ref[...], preferred_element_type=jnp.float32)
    adst = jnp.dot(h, ad_ref[...], preferred_element_type=jnp.float32)
    asrc_ref[...] = asrc
    adst_ref[...] = adst
    bs = jnp.max(asrc, axis=0, keepdims=True)
    bd = jnp.max(adst, axis=0, keepdims=True)

    @pl.when(i == 0)
    def _():
        ms_ref[...] = bs
        md_ref[...] = bd

    @pl.when(i > 0)
    def _():
        ms_ref[...] = jnp.maximum(ms_ref[...], bs)
        md_ref[...] = jnp.maximum(md_ref[...], bd)

    @pl.when(i == NG - 1)
    def _():
        ks = ms_ref[...] + md_ref[...]
        k_ref[...] = jnp.where(ks >= 0.0, ks, 0.2 * ks)


def _phase_a(x, w, a_src_m, a_dst_m):
    return pl.pallas_call(
        _phase_a_body,
        grid=(NG,),
        in_specs=[
            pl.BlockSpec((BN, D), lambda i: (i, 0)),
            pl.BlockSpec((D, D), lambda i: (0, 0)),
            pl.BlockSpec((D, H), lambda i: (0, 0)),
            pl.BlockSpec((D, H), lambda i: (0, 0)),
        ],
        out_specs=[
            pl.BlockSpec((BN, D), lambda i: (i, 0)),
            pl.BlockSpec((BN, H), lambda i: (i, 0)),
            pl.BlockSpec((BN, H), lambda i: (i, 0)),
            pl.BlockSpec((1, H), lambda i: (0, 0)),
        ],
        out_shape=[
            jax.ShapeDtypeStruct((N, D), jnp.float32),
            jax.ShapeDtypeStruct((N, H), jnp.float32),
            jax.ShapeDtypeStruct((N, H), jnp.float32),
            jax.ShapeDtypeStruct((1, H), jnp.float32),
        ],
        scratch_shapes=[
            pltpu.VMEM((1, H), jnp.float32),
            pltpu.VMEM((1, H), jnp.float32),
        ],
    )(x, w, a_src_m, a_dst_m)


def _splat(vec, lane):
    idx = jnp.full((16,), lane, dtype=jnp.int32)
    dn = lax.GatherDimensionNumbers(
        offset_dims=(), collapsed_slice_dims=(0,), start_index_map=(0,))
    return lax.gather(vec, idx[:, None], dn, (1,),
                      mode=lax.GatherScatterMode.PROMISE_IN_BOUNDS)


def _pass1_body(asrc_hbm, adst_hbm, k_hbm, src_hbm, dst_hbm, p_hbm, den_hbm,
                asrc_v, adst_v, k_v, srcb, dstb, pblk, dloc):
    cid = lax.axis_index("c")
    sid = lax.axis_index("s")
    wid = cid * 16 + sid
    base0 = wid * CHUNK
    zeros16 = jnp.zeros((16,), jnp.float32)
    lanes = lax.iota(jnp.int32, 16)

    pltpu.sync_copy(asrc_hbm, asrc_v)
    pltpu.sync_copy(adst_hbm, adst_v)
    pltpu.sync_copy(k_hbm, k_v)

    def zloc(i, _):
        dloc[pl.ds(i * 16, 16)] = zeros16
        return 0

    lax.fori_loop(0, N * H // 16, zloc, 0)

    kb = [_splat(k_v[...], h) for h in range(H)]

    def block(b, _):
        base = base0 + b * BE1
        pltpu.sync_copy(src_hbm.at[pl.ds(base, BE1)], srcb)
        pltpu.sync_copy(dst_hbm.at[pl.ds(base, BE1)], dstb)

        def group(g, _):
            sv = srcb[pl.ds(g * 16, 16)]
            dv = jnp.minimum(dstb[pl.ds(g * 16, 16)], N - 1)
            gid = base + g * 16 + lanes
            valid = gid < ET
            rec = (g * 16 + lanes) * EW
            plsc.store_scatter(pblk, [rec],
                               plsc.bitcast(sv, jnp.float32))
            plsc.store_scatter(pblk, [rec + 1],
                               plsc.bitcast(dstb[pl.ds(g * 16, 16)],
                                            jnp.float32))
            for h in range(H):
                av = plsc.load_gather(asrc_v, [sv * H + h])
                bv = plsc.load_gather(adst_v, [dv * H + h])
                e = av + bv
                e = jnp.where(e >= 0.0, e, 0.2 * e) - kb[h]
                p = jnp.where(valid, jnp.exp(e), 0.0)
                plsc.store_scatter(pblk, [rec + 2 + h], p)
                plsc.addupdate_scatter(dloc, [dv * H + h], p)
            return 0

        lax.fori_loop(0, BE1 // 16, group, 0)
        pltpu.sync_copy(pblk, p_hbm.at[pl.ds(base * EW, BE1 * EW)])
        return 0

    lax.fori_loop(0, NBLK1, block, 0)

    pltpu.sync_copy(dloc, den_hbm.at[wid])


def _pass1(asrc, adst, k, src, dst):
    mesh = plsc.VectorSubcoreMesh(core_axis_name="c", subcore_axis_name="s",
                                  num_cores=2, num_subcores=16)
    f = pl.kernel(
        _pass1_body,
        out_type=(
            jax.ShapeDtypeStruct((EPAD * EW,), jnp.float32),
            jax.ShapeDtypeStruct((NTILES, N * H), jnp.float32),
        ),
        mesh=mesh,
        scratch_types=[
            pltpu.VMEM((N * H,), jnp.float32),
            pltpu.VMEM((N * H,), jnp.float32),
            pltpu.VMEM((16,), jnp.float32),
            pltpu.VMEM((BE1,), jnp.int32),
            pltpu.VMEM((BE1,), jnp.int32),
            pltpu.VMEM((BE1 * EW,), jnp.float32),
            pltpu.VMEM((N * H,), jnp.float32),
        ],
        compiler_params=pltpu.CompilerParams(needs_layout_passes=False),
    )
    return f(asrc, adst, k, src, dst)


def _pass2_body(h_hbm, ed_hbm, out_hbm,
                hrows0, hrows1, eb0, eb1, srcb0, srcb1, dstb0, dstb1, acc,
                gsem0, gsem1, esem0, esem1, ssem0, ssem1):
    cid = lax.axis_index("c")
    sid = lax.axis_index("s")
    wid = cid * 16 + sid
    base0 = wid * CHUNK
    zeros16 = jnp.zeros((16,), jnp.float32)
    lanes = lax.iota(jnp.int32, 16)
    qoff = (lanes >> 2) * EW + 2 + (lanes & 3)

    hrows = [hrows0, hrows1]
    eb = [eb0, eb1]
    srcb = [srcb0, srcb1]
    dstb = [dstb0, dstb1]
    gsem = [gsem0, gsem1]
    esem = [esem0, esem1]
    ssem = [ssem0, ssem1]

    def zrow(r, _):
        for k in range(D // 16):
            hrows0[r, pl.ds(k * 16, 16)] = zeros16
        return 0

    lax.fori_loop(0, BE, zrow, 0)
    for k in range(ROWS_PER_TILE // BE):
        pltpu.sync_copy(hrows0,
                        acc.at[pl.ds(sid * ROWS_PER_TILE + k * BE, BE)])
    plsc.subcore_barrier()

    def fetch_ed(b, q):
        base = base0 + b * BE
        pltpu.async_copy(ed_hbm.at[pl.ds(base * EW, BE * EW)], eb[q],
                         esem[q])

    def extract_idx(q):
        for g in range(BE // 16):
            rec = (g * 16 + lanes) * EW
            sv = plsc.load_gather(eb[q], [rec])
            dv = plsc.load_gather(eb[q], [rec + 1])
            srcb[q][pl.ds(g * 16, 16)] = plsc.bitcast(sv, jnp.int32)
            dstb[q][pl.ds(g * 16, 16)] = plsc.bitcast(dv, jnp.int32)

    def compute(q):
        def quad(i, _):
            pvec = plsc.load_gather(eb[q], [qoff + i * (4 * EW)])
            for j in range(4):
                e = i * 4 + j
                for h in range(H):
                    s = _splat(pvec, j * H + h)
                    for sub in range(2):
                        cg = h * 2 + sub
                        hrows[q][e, pl.ds(cg * 16, 16)] = (
                            hrows[q][e, pl.ds(cg * 16, 16)] * s)
            return 0

        lax.fori_loop(0, BE // 4, quad, 0)

    fetch_ed(0, 0)
    pltpu.make_async_copy(ed_hbm.at[pl.ds(base0 * EW, BE * EW)], eb[0],
                          esem[0]).wait()
    extract_idx(0)
    pltpu.async_copy(h_hbm.at[srcb[0]], hrows[0], gsem[0])
    fetch_ed(1, 1)

    def pair(t, _):
        for k in range(2):
            b = 2 * t + k
            q = k
            r = 1 - k
            pltpu.make_async_copy(h_hbm.at[srcb[q]], hrows[q],
                                  gsem[q]).wait()
            pltpu.make_async_copy(ed_hbm.at[pl.ds(0, BE * EW)], eb[r],
                                  esem[r]).wait()
            if k == 1:
                pltpu.make_async_copy(hrows[r], acc.at[dstb[r]],
                                      ssem[r]).wait()
            else:
                @pl.when(t > 0)
                def _():
                    pltpu.make_async_copy(hrows[r], acc.at[dstb[r]],
                                          ssem[r]).wait()
            extract_idx(r)
            pltpu.async_copy(h_hbm.at[srcb[r]], hrows[r], gsem[r])
            compute(q)
            bn = jnp.minimum(b + 2, NBLK - 1)
            fetch_ed(bn, q)
            pltpu.async_copy(hrows[q], acc.at[dstb[q]], ssem[q], add=True)
        return 0

    lax.fori_loop(0, NBLK // 2, pair, 0)

    pltpu.make_async_copy(h_hbm.at[srcb[0]], hrows[0], gsem[0]).wait()
    pltpu.make_async_copy(ed_hbm.at[pl.ds(0, BE * EW)], eb[1],
                          esem[1]).wait()
    pltpu.make_async_copy(hrows[1], acc.at[dstb[1]], ssem[1]).wait()

    plsc.subcore_barrier()
    pltpu.sync_copy(acc.at[pl.ds(sid * ROWS_PER_TILE, ROWS_PER_TILE)],
                    out_hbm.at[cid, pl.ds(sid * ROWS_PER_TILE, ROWS_PER_TILE)])


def _pass2(h, ed):
    mesh = plsc.VectorSubcoreMesh(core_axis_name="c", subcore_axis_name="s",
                                  num_cores=2, num_subcores=16)
    f = pl.kernel(
        _pass2_body,
        out_type=jax.ShapeDtypeStruct((2, NPAD, D), jnp.float32),
        mesh=mesh,
        scratch_types=[
            pltpu.VMEM((BE, D), jnp.float32),
            pltpu.VMEM((BE, D), jnp.float32),
            pltpu.VMEM((BE * EW,), jnp.float32),
            pltpu.VMEM((BE * EW,), jnp.float32),
            pltpu.VMEM((BE,), jnp.int32),
            pltpu.VMEM((BE,), jnp.int32),
            pltpu.VMEM((BE,), jnp.int32),
            pltpu.VMEM((BE,), jnp.int32),
            pltpu.VMEM_SHARED((NPAD, D), jnp.float32),
            pltpu.SemaphoreType.DMA,
            pltpu.SemaphoreType.DMA,
            pltpu.SemaphoreType.DMA,
            pltpu.SemaphoreType.DMA,
            pltpu.SemaphoreType.DMA,
            pltpu.SemaphoreType.DMA,
        ],
        compiler_params=pltpu.CompilerParams(needs_layout_passes=False),
    )
    return f(h, ed)


def _phase_d_body(part_ref, den_ref, x_ref, s_ref, bias_ref, lnw_ref, lnb_ref,
                  out_ref):
    num = part_ref[0] + part_ref[1]
    den4 = jnp.sum(den_ref[...], axis=0)
    den = jnp.dot(den4, s_ref[...], preferred_element_type=jnp.float32)
    g = num / den + bias_ref[...]
    mean = jnp.mean(g, axis=1, keepdims=True)
    cen = g - mean
    var = jnp.mean(cen * cen, axis=1, keepdims=True)
    norm = cen / jnp.sqrt(var + 1e-12)
    out_ref[...] = x_ref[...] + norm * lnw_ref[...] + lnb_ref[...]


def _phase_d(part, den, x, s, bias, lnw, lnb):
    return pl.pallas_call(
        _phase_d_body,
        grid=(NG,),
        in_specs=[
            pl.BlockSpec((2, BN, D), lambda i: (0, i, 0)),
            pl.BlockSpec((NTILES, BN, H), lambda i: (0, i, 0)),
            pl.BlockSpec((BN, D), lambda i: (i, 0)),
            pl.BlockSpec((H, D), lambda i: (0, 0)),
            pl.BlockSpec((1, D), lambda i: (0, 0)),
            pl.BlockSpec((1, D), lambda i: (0, 0)),
            pl.BlockSpec((1, D), lambda i: (0, 0)),
        ],
        out_specs=pl.BlockSpec((BN, D), lambda i: (i, 0)),
        out_shape=jax.ShapeDtypeStruct((N, D), jnp.float32),
    )(part, den, x, s, bias, lnw, lnb)


def kernel(x, edge_index, W, att_src, att_dst, bias, ln_weight, ln_bias):
    f32 = jnp.float32
    blk = jnp.repeat(jnp.eye(H, dtype=f32), C, axis=0)
    a_src_m = blk * att_src.reshape(D, 1)
    a_dst_m = blk * att_dst.reshape(D, 1)
    s_exp = jnp.repeat(jnp.eye(H, dtype=f32), C, axis=1)

    loops = jnp.arange(N, dtype=edge_index.dtype)
    pad_ar = jnp.arange(EPAD - ET, dtype=edge_index.dtype)
    pad_dst = N + pad_ar % (NPAD - N)
    pad_src = pad_ar % N
    src = jnp.concatenate([edge_index[0], loops, pad_src])
    dst = jnp.concatenate([edge_index[1], loops, pad_dst])

    h, asrc, adst, k = _phase_a(x, W, a_src_m, a_dst_m)
    kpad = jnp.concatenate([k.reshape(-1), jnp.zeros((16 - H,), f32)])
    ed, den = _pass1(asrc.reshape(-1), adst.reshape(-1), kpad, src, dst)
    part = _pass2(h, ed)
    return _phase_d(part, den.reshape(NTILES, N, H), x, s_exp,
                    bias.reshape(1, D), ln_weight.reshape(1, D),
                    ln_bias.reshape(1, D))


# speedup vs baseline: 1.1681x; 1.1681x over previous
import jax
import jax.numpy as jnp
from jax import lax
from jax.experimental import pallas as pl
from jax.experimental.pallas import tpu as pltpu
from jax.experimental.pallas import tpu_sc as plsc

N = 10000
E = 320000
D = 128
H = 4
C = D // H

ET = E + N
NTILES = 32
BE = 128
CHUNK = 10752
EPAD = CHUNK * NTILES
NBLK = CHUNK // BE
BE1 = CHUNK // 16
NBLK1 = CHUNK // BE1
EW = 8
NPAD = 10240
ROWS_PER_TILE = NPAD // 16

BN = 1000
NG = N // BN


def _phase_a_body(x_ref, w_ref, as_ref, ad_ref, h_ref, asrc_ref, adst_ref,
                  k_ref, ms_ref, md_ref):
    i = pl.program_id(0)
    h = jnp.dot(x_ref[...], w_ref[...], preferred_element_type=jnp.float32)
    h_ref[...] = h
    asrc = jnp.dot(h, as_ref[...], preferred_element_type=jnp.float32)
    adst = jnp.dot(h, ad_ref[...], preferred_element_type=jnp.float32)
    asrc_ref[...] = asrc
    adst_ref[...] = adst
    bs = jnp.max(asrc, axis=0, keepdims=True)
    bd = jnp.max(adst, axis=0, keepdims=True)

    @pl.when(i == 0)
    def _():
        ms_ref[...] = bs
        md_ref[...] = bd

    @pl.when(i > 0)
    def _():
        ms_ref[...] = jnp.maximum(ms_ref[...], bs)
        md_ref[...] = jnp.maximum(md_ref[...], bd)

    @pl.when(i == NG - 1)
    def _():
        ks = ms_ref[...] + md_ref[...]
        k_ref[...] = jnp.where(ks >= 0.0, ks, 0.2 * ks)


def _phase_a(x, w, a_src_m, a_dst_m):
    return pl.pallas_call(
        _phase_a_body,
        grid=(NG,),
        in_specs=[
            pl.BlockSpec((BN, D), lambda i: (i, 0)),
            pl.BlockSpec((D, D), lambda i: (0, 0)),
            pl.BlockSpec((D, H), lambda i: (0, 0)),
            pl.BlockSpec((D, H), lambda i: (0, 0)),
        ],
        out_specs=[
            pl.BlockSpec((BN, D), lambda i: (i, 0)),
            pl.BlockSpec((BN, H), lambda i: (i, 0)),
            pl.BlockSpec((BN, H), lambda i: (i, 0)),
            pl.BlockSpec((1, H), lambda i: (0, 0)),
        ],
        out_shape=[
            jax.ShapeDtypeStruct((N, D), jnp.float32),
            jax.ShapeDtypeStruct((N, H), jnp.float32),
            jax.ShapeDtypeStruct((N, H), jnp.float32),
            jax.ShapeDtypeStruct((1, H), jnp.float32),
        ],
        scratch_shapes=[
            pltpu.VMEM((1, H), jnp.float32),
            pltpu.VMEM((1, H), jnp.float32),
        ],
    )(x, w, a_src_m, a_dst_m)


def _splat(vec, lane):
    idx = jnp.full((16,), lane, dtype=jnp.int32)
    dn = lax.GatherDimensionNumbers(
        offset_dims=(), collapsed_slice_dims=(0,), start_index_map=(0,))
    return lax.gather(vec, idx[:, None], dn, (1,),
                      mode=lax.GatherScatterMode.PROMISE_IN_BOUNDS)


def _pass1_body(asrc_hbm, adst_hbm, k_hbm, src_hbm, dst_hbm, p_hbm, den_hbm,
                asrc_v, adst_v, k_v, srcb, dstb, pblk, dloc):
    cid = lax.axis_index("c")
    sid = lax.axis_index("s")
    wid = cid * 16 + sid
    base0 = wid * CHUNK
    zeros16 = jnp.zeros((16,), jnp.float32)
    lanes = lax.iota(jnp.int32, 16)

    pltpu.sync_copy(asrc_hbm, asrc_v)
    pltpu.sync_copy(adst_hbm, adst_v)
    pltpu.sync_copy(k_hbm, k_v)

    def zloc(i, _):
        dloc[pl.ds(i * 16, 16)] = zeros16
        return 0

    lax.fori_loop(0, N * H // 16, zloc, 0)

    kb = [_splat(k_v[...], h) for h in range(H)]

    def block(b, _):
        base = base0 + b * BE1
        pltpu.sync_copy(src_hbm.at[pl.ds(base, BE1)], srcb)
        pltpu.sync_copy(dst_hbm.at[pl.ds(base, BE1)], dstb)

        def group(g, _):
            sv = srcb[pl.ds(g * 16, 16)]
            dv = jnp.minimum(dstb[pl.ds(g * 16, 16)], N - 1)
            gid = base + g * 16 + lanes
            valid = gid < ET
            rec = (g * 16 + lanes) * EW
            plsc.store_scatter(pblk, [rec],
                               plsc.bitcast(sv, jnp.float32))
            plsc.store_scatter(pblk, [rec + 1],
                               plsc.bitcast(dstb[pl.ds(g * 16, 16)],
                                            jnp.float32))
            for h in range(H):
                av = plsc.load_gather(asrc_v, [sv * H + h])
                bv = plsc.load_gather(adst_v, [dv * H + h])
                e = av + bv
                e = jnp.where(e >= 0.0, e, 0.2 * e) - kb[h]
                p = jnp.where(valid, jnp.exp(e), 0.0)
                plsc.store_scatter(pblk, [rec + 2 + h], p)
                plsc.addupdate_scatter(dloc, [dv * H + h], p)
            return 0

        lax.fori_loop(0, BE1 // 16, group, 0)
        pltpu.sync_copy(pblk, p_hbm.at[pl.ds(base * EW, BE1 * EW)])
        return 0

    lax.fori_loop(0, NBLK1, block, 0)

    pltpu.sync_copy(dloc, den_hbm.at[wid])


def _pass1(asrc, adst, k, src, dst):
    mesh = plsc.VectorSubcoreMesh(core_axis_name="c", subcore_axis_name="s",
                                  num_cores=2, num_subcores=16)
    f = pl.kernel(
        _pass1_body,
        out_type=(
            jax.ShapeDtypeStruct((EPAD * EW,), jnp.float32),
            jax.ShapeDtypeStruct((NTILES, N * H), jnp.float32),
        ),
        mesh=mesh,
        scratch_types=[
            pltpu.VMEM((N * H,), jnp.float32),
            pltpu.VMEM((N * H,), jnp.float32),
            pltpu.VMEM((16,), jnp.float32),
            pltpu.VMEM((BE1,), jnp.int32),
            pltpu.VMEM((BE1,), jnp.int32),
            pltpu.VMEM((BE1 * EW,), jnp.float32),
            pltpu.VMEM((N * H,), jnp.float32),
        ],
        compiler_params=pltpu.CompilerParams(needs_layout_passes=False),
    )
    return f(asrc, adst, k, src, dst)


def _pass2_body(h_hbm, ed_hbm, out_hbm,
                hrows0, hrows1, eb0, eb1, srcb0, srcb1, dstb0, dstb1, acc,
                gsem0, gsem1, esem0, esem1, ssem0, ssem1):
    cid = lax.axis_index("c")
    sid = lax.axis_index("s")
    wid = cid * 16 + sid
    base0 = wid * CHUNK
    zeros16 = jnp.zeros((16,), jnp.float32)
    lanes = lax.iota(jnp.int32, 16)
    qoff = (lanes >> 2) * EW + 2 + (lanes & 3)

    hrows = [hrows0, hrows1]
    eb = [eb0, eb1]
    srcb = [srcb0, srcb1]
    dstb = [dstb0, dstb1]
    gsem = [gsem0, gsem1]
    esem = [esem0, esem1]
    ssem = [ssem0, ssem1]

    def zrow(r, _):
        for k in range(D // 16):
            hrows0[r, pl.ds(k * 16, 16)] = zeros16
        return 0

    lax.fori_loop(0, BE, zrow, 0)
    for k in range(ROWS_PER_TILE // BE):
        pltpu.sync_copy(hrows0,
                        acc.at[pl.ds(sid * ROWS_PER_TILE + k * BE, BE)])
    plsc.subcore_barrier()

    def fetch_ed(b, q):
        base = base0 + b * BE
        pltpu.async_copy(ed_hbm.at[pl.ds(base * EW, BE * EW)], eb[q],
                         esem[q])

    def extract_idx(q):
        for g in range(BE // 16):
            rec = (g * 16 + lanes) * EW
            sv = plsc.load_gather(eb[q], [rec])
            dv = plsc.load_gather(eb[q], [rec + 1])
            srcb[q][pl.ds(g * 16, 16)] = plsc.bitcast(sv, jnp.int32)
            dstb[q][pl.ds(g * 16, 16)] = plsc.bitcast(dv, jnp.int32)

    def compute(q):
        def quad(i, _):
            pvec = plsc.load_gather(eb[q], [qoff + i * (4 * EW)])
            for j in range(4):
                e = i * 4 + j
                for h in range(H):
                    s = _splat(pvec, j * H + h)
                    for sub in range(2):
                        cg = h * 2 + sub
                        hrows[q][e, pl.ds(cg * 16, 16)] = (
                            hrows[q][e, pl.ds(cg * 16, 16)] * s)
            return 0

        lax.fori_loop(0, BE // 4, quad, 0)

    fetch_ed(0, 0)
    pltpu.make_async_copy(ed_hbm.at[pl.ds(base0 * EW, BE * EW)], eb[0],
                          esem[0]).wait()
    extract_idx(0)
    pltpu.async_copy(h_hbm.at[srcb[0]], hrows[0], gsem[0])
    fetch_ed(1, 1)

    def pair(t, _):
        for k in range(2):
            b = 2 * t + k
            q = k
            r = 1 - k
            pltpu.make_async_copy(h_hbm.at[srcb[q]], hrows[q],
                                  gsem[q]).wait()
            pltpu.make_async_copy(ed_hbm.at[pl.ds(0, BE * EW)], eb[r],
                                  esem[r]).wait()
            if k == 1:
                pltpu.make_async_copy(hrows[r], acc.at[dstb[r]],
                                      ssem[r]).wait()
            else:
                @pl.when(t > 0)
                def _():
                    pltpu.make_async_copy(hrows[r], acc.at[dstb[r]],
                                          ssem[r]).wait()
            extract_idx(r)
            pltpu.async_copy(h_hbm.at[srcb[r]], hrows[r], gsem[r])
            compute(q)
            bn = jnp.minimum(b + 2, NBLK - 1)
            fetch_ed(bn, q)
            pltpu.async_copy(hrows[q], acc.at[dstb[q]], ssem[q], add=True)
        return 0

    lax.fori_loop(0, NBLK // 2, pair, 0)

    pltpu.make_async_copy(h_hbm.at[srcb[0]], hrows[0], gsem[0]).wait()
    pltpu.make_async_copy(ed_hbm.at[pl.ds(0, BE * EW)], eb[1],
                          esem[1]).wait()
    pltpu.make_async_copy(hrows[1], acc.at[dstb[1]], ssem[1]).wait()

    plsc.subcore_barrier()
    pltpu.sync_copy(acc.at[pl.ds(sid * ROWS_PER_TILE, ROWS_PER_TILE)],
                    out_hbm.at[cid, pl.ds(sid * ROWS_PER_TILE, ROWS_PER_TILE)])


def _pass2(h, ed):
    mesh = plsc.VectorSubcoreMesh(core_axis_name="c", subcore_axis_name="s",
                                  num_cores=2, num_subcores=16)
    f = pl.kernel(
        _pass2_body,
        out_type=jax.ShapeDtypeStruct((2, NPAD, D), jnp.float32),
        mesh=mesh,
        scratch_types=[
            pltpu.VMEM((BE, D), jnp.float32),
            pltpu.VMEM((BE, D), jnp.float32),
            pltpu.VMEM((BE * EW,), jnp.float32),
            pltpu.VMEM((BE * EW,), jnp.float32),
            pltpu.VMEM((BE,), jnp.int32),
            pltpu.VMEM((BE,), jnp.int32),
            pltpu.VMEM((BE,), jnp.int32),
            pltpu.VMEM((BE,), jnp.int32),
            pltpu.VMEM_SHARED((NPAD, D), jnp.float32),
            pltpu.SemaphoreType.DMA,
            pltpu.SemaphoreType.DMA,
            pltpu.SemaphoreType.DMA,
            pltpu.SemaphoreType.DMA,
            pltpu.SemaphoreType.DMA,
            pltpu.SemaphoreType.DMA,
        ],
        compiler_params=pltpu.CompilerParams(needs_layout_passes=False),
    )
    return f(h, ed)


def _phase_d_body(part_ref, den_ref, x_ref, ms_ref, bias_ref, lnw_ref,
                  lnb_ref, out_ref):
    num = part_ref[0] + part_ref[1]
    den = jnp.dot(den_ref[...], ms_ref[...],
                  preferred_element_type=jnp.float32)
    g = num / den + bias_ref[...]
    mean = jnp.mean(g, axis=1, keepdims=True)
    cen = g - mean
    var = jnp.mean(cen * cen, axis=1, keepdims=True)
    norm = cen / jnp.sqrt(var + 1e-12)
    out_ref[...] = x_ref[...] + norm * lnw_ref[...] + lnb_ref[...]


def _phase_d(part, den, x, ms, bias, lnw, lnb):
    return pl.pallas_call(
        _phase_d_body,
        grid=(NG,),
        in_specs=[
            pl.BlockSpec((2, BN, D), lambda i: (0, i, 0)),
            pl.BlockSpec((BN, NTILES * H), lambda i: (i, 0)),
            pl.BlockSpec((BN, D), lambda i: (i, 0)),
            pl.BlockSpec((NTILES * H, D), lambda i: (0, 0)),
            pl.BlockSpec((1, D), lambda i: (0, 0)),
            pl.BlockSpec((1, D), lambda i: (0, 0)),
            pl.BlockSpec((1, D), lambda i: (0, 0)),
        ],
        out_specs=pl.BlockSpec((BN, D), lambda i: (i, 0)),
        out_shape=jax.ShapeDtypeStruct((N, D), jnp.float32),
    )(part, den, x, ms, bias, lnw, lnb)


def kernel(x, edge_index, W, att_src, att_dst, bias, ln_weight, ln_bias):
    f32 = jnp.float32
    blk = jnp.repeat(jnp.eye(H, dtype=f32), C, axis=0)
    a_src_m = blk * att_src.reshape(D, 1)
    a_dst_m = blk * att_dst.reshape(D, 1)
    s_exp = jnp.repeat(jnp.eye(H, dtype=f32), C, axis=1)
    ms_fold = jnp.tile(s_exp, (NTILES, 1))

    loops = jnp.arange(N, dtype=edge_index.dtype)
    pad_ar = jnp.arange(EPAD - ET, dtype=edge_index.dtype)
    pad_dst = N + pad_ar % (NPAD - N)
    pad_src = pad_ar % N
    src = jnp.concatenate([edge_index[0], loops, pad_src])
    dst = jnp.concatenate([edge_index[1], loops, pad_dst])

    h, asrc, adst, k = _phase_a(x, W, a_src_m, a_dst_m)
    kpad = jnp.concatenate([k.reshape(-1), jnp.zeros((16 - H,), f32)])
    ed, den = _pass1(asrc.reshape(-1), adst.reshape(-1), kpad, src, dst)
    part = _pass2(h, ed)
    den_t = den.reshape(NTILES, N, H).transpose(1, 0, 2).reshape(N,
                                                                 NTILES * H)
    return _phase_d(part, den_t, x, ms_fold,
                    bias.reshape(1, D), ln_weight.reshape(1, D),
                    ln_bias.reshape(1, D))
